# Initial kernel scaffold; baseline (speedup 1.0000x reference)
import jax
import jax.numpy as jnp
from jax.experimental import pallas as pl


def kernel(x, edge_index, W1l, b1l, W1r, W2l, b2l, W2r, W3, b3):
    raise NotImplementedError("write your pallas kernel here")


# speedup vs baseline: 2.8153x; 2.8153x over previous
import functools

import jax
import jax.numpy as jnp
from jax import lax
from jax.experimental import pallas as pl
from jax.experimental.pallas import tpu as pltpu
from jax.experimental.pallas import tpu_sc as plsc

N = 10000
N_PAD = 10240
E = 160000
E_PAD = 163840
ROWS = E_PAD // 128
ROWS_PER_TILE = ROWS // 16
ACC_ROWS_PER_TILE = N_PAD // 16
F = 128


def _sc_body(with_deg, t0, t1, srcr, dstr, zros, acc0_out, acc1_out, *rest):
  if with_deg:
    deg_out, acc, srcv, dstv, buf, degv, sem = rest
  else:
    acc, srcv, dstv, buf, sem = rest
    degv = None
  c = lax.axis_index("c")
  s = lax.axis_index("s")

  ibase = s * ROWS_PER_TILE
  pltpu.sync_copy(srcr.at[pl.ds(ibase, ROWS_PER_TILE)], srcv)
  pltpu.sync_copy(dstr.at[pl.ds(ibase, ROWS_PER_TILE)], dstv)

  rbase = s * ACC_ROWS_PER_TILE
  pltpu.sync_copy(zros.at[pl.ds(rbase, ACC_ROWS_PER_TILE)],
                  acc.at[pl.ds(rbase, ACC_ROWS_PER_TILE)])
  if with_deg:
    zeros16 = jnp.zeros((16,), jnp.float32)

    @pl.when(c == 0)
    def _():
      def zstep(j, carry):
        degv[pl.ds(j * 16, 16)] = zeros16
        return carry
      lax.fori_loop(0, N_PAD // 16, zstep, 0)

  plsc.subcore_barrier()

  ones16 = jnp.ones((16,), jnp.float32)

  def _process(table, count_deg):
    def step(j, carry):
      pltpu.async_copy(table.at[srcv.at[j]], buf, sem).wait()
      pltpu.sync_copy(buf, acc.at[dstv.at[j]], add=True)
      if count_deg:
        for k in range(8):
          idx = dstv[j, pl.ds(k * 16, 16)]
          plsc.addupdate_scatter(degv, [idx], ones16)
      return carry
    lax.fori_loop(0, ROWS_PER_TILE, step, 0)

  @pl.when(c == 0)
  def _():
    _process(t0, with_deg)

  @pl.when(c == 1)
  def _():
    _process(t1, False)

  plsc.subcore_barrier()

  sl = pl.ds(rbase, ACC_ROWS_PER_TILE)

  @pl.when(c == 0)
  def _():
    pltpu.sync_copy(acc.at[sl], acc0_out.at[sl])
    if with_deg:
      pltpu.sync_copy(degv, deg_out.at[s])

  @pl.when(c == 1)
  def _():
    pltpu.sync_copy(acc.at[sl], acc1_out.at[sl])


@functools.lru_cache(maxsize=None)
def _make_sc_seg(with_deg):
  mesh = plsc.VectorSubcoreMesh(core_axis_name="c", subcore_axis_name="s",
                                num_cores=2, num_subcores=16)
  out_type = [jax.ShapeDtypeStruct((N_PAD, F), jnp.float32),
              jax.ShapeDtypeStruct((N_PAD, F), jnp.float32)]
  if with_deg:
    out_type += [jax.ShapeDtypeStruct((16, N_PAD), jnp.float32)]
  scratch = [
      pltpu.VMEM_SHARED((N_PAD, F), jnp.float32),
      pltpu.VMEM((ROWS_PER_TILE, 128), jnp.int32),
      pltpu.VMEM((ROWS_PER_TILE, 128), jnp.int32),
      pltpu.VMEM((128, F), jnp.float32),
  ]
  if with_deg:
    scratch += [pltpu.VMEM((N_PAD,), jnp.float32)]
  scratch += [pltpu.SemaphoreType.DMA]
  return pl.kernel(
      functools.partial(_sc_body, with_deg),
      out_type=tuple(out_type),
      mesh=mesh,
      scratch_types=tuple(scratch),
      compiler_params=pltpu.CompilerParams(needs_layout_passes=False),
  )


def _dense1_body(agg0, agg1, degp, x_ref, w1l, w1r, b1, out):
  deg = jnp.sum(degp[...], axis=1)[:, None]
  r = 1.0 / jnp.maximum(deg, 1.0)
  h = jnp.dot(agg0[...] * r, w1l[0:128, :], preferred_element_type=jnp.float32)
  h += jnp.dot(agg1[...] * r, w1l[128:256, :], preferred_element_type=jnp.float32)
  h += jnp.dot(x_ref[...], w1r[...], preferred_element_type=jnp.float32)
  h = jnp.maximum(h + b1[...], 0.0)
  for q in range(4):
    out[q, :, :] = h[:, 128 * q:128 * (q + 1)]


def _dense2_body(a0, a1, a2, a3, degp, h1, w2l, w2r, b2, w3, b3, out):
  deg = jnp.sum(degp[...], axis=1)[:, None]
  r = 1.0 / jnp.maximum(deg, 1.0)
  aggs = (a0, a1, a2, a3)
  h = b2[...] + jnp.zeros((a0.shape[0], 512), jnp.float32)
  for q in range(4):
    h += jnp.dot(aggs[q][...] * r, w2l[128 * q:128 * (q + 1), :],
                 preferred_element_type=jnp.float32)
    h += jnp.dot(h1[q, :, :], w2r[128 * q:128 * (q + 1), :],
                 preferred_element_type=jnp.float32)
  h = jnp.maximum(h, 0.0)
  out[...] = jnp.dot(h, w3[...], preferred_element_type=jnp.float32) + b3[...]


_MB = 1000


def _dense1(agg0, agg1, degp, x, W1l, W1r, b1):
  grid = (N // _MB,)
  return pl.pallas_call(
      _dense1_body,
      grid=grid,
      in_specs=[
          pl.BlockSpec((_MB, F), lambda i: (i, 0)),
          pl.BlockSpec((_MB, F), lambda i: (i, 0)),
          pl.BlockSpec((_MB, 16), lambda i: (i, 0)),
          pl.BlockSpec((_MB, 256), lambda i: (i, 0)),
          pl.BlockSpec((256, 512), lambda i: (0, 0)),
          pl.BlockSpec((256, 512), lambda i: (0, 0)),
          pl.BlockSpec((1, 512), lambda i: (0, 0)),
      ],
      out_specs=pl.BlockSpec((4, _MB, 128), lambda i: (0, i, 0)),
      out_shape=jax.ShapeDtypeStruct((4, N, 128), jnp.float32),
  )(agg0, agg1, degp, x, W1l, W1r, b1)


def _dense2(a0, a1, a2, a3, degp, h1t, W2l, W2r, b2, W3, b3):
  grid = (N // _MB,)
  agg_spec = pl.BlockSpec((_MB, F), lambda i: (i, 0))
  return pl.pallas_call(
      _dense2_body,
      grid=grid,
      in_specs=[
          agg_spec, agg_spec, agg_spec, agg_spec,
          pl.BlockSpec((_MB, 16), lambda i: (i, 0)),
          pl.BlockSpec((4, _MB, 128), lambda i: (0, i, 0)),
          pl.BlockSpec((512, 512), lambda i: (0, 0)),
          pl.BlockSpec((512, 512), lambda i: (0, 0)),
          pl.BlockSpec((1, 512), lambda i: (0, 0)),
          pl.BlockSpec((512, 256), lambda i: (0, 0)),
          pl.BlockSpec((1, 256), lambda i: (0, 0)),
      ],
      out_specs=pl.BlockSpec((_MB, 256), lambda i: (i, 0)),
      out_shape=jax.ShapeDtypeStruct((N, 256), jnp.float32),
  )(a0, a1, a2, a3, degp, h1t, W2l, W2r, b2, W3, b3)


def kernel(x, edge_index, W1l, b1l, W1r, W2l, b2l, W2r, W3, b3):
  x = x.astype(jnp.float32)
  src = edge_index[0].astype(jnp.int32)
  dst = edge_index[1].astype(jnp.int32)
  pad = E_PAD - E
  src2d = jnp.concatenate([src, jnp.zeros((pad,), jnp.int32)]).reshape(ROWS, 128)
  dst2d = jnp.concatenate([dst, jnp.full((pad,), N, jnp.int32)]).reshape(ROWS, 128)
  zeros = jnp.zeros((N_PAD, 128), jnp.float32)

  t0 = x[:, :128]
  t1 = x[:, 128:]
  agg0, agg1, degp = _make_sc_seg(True)(t0, t1, src2d, dst2d, zeros)
  degp = degp.T

  h1t = _dense1(agg0, agg1, degp, x, W1l, W1r, b1l.reshape(1, 512))

  a0, a1 = _make_sc_seg(False)(h1t[0], h1t[1], src2d, dst2d, zeros)
  a2, a3 = _make_sc_seg(False)(h1t[2], h1t[3], src2d, dst2d, zeros)

  return _dense2(a0, a1, a2, a3, degp, h1t,
                 W2l, W2r, b2l.reshape(1, 512), W3, b3.reshape(1, 256))


# speedup vs baseline: 3.3082x; 1.1751x over previous
import functools

import jax
import jax.numpy as jnp
from jax import lax
from jax.experimental import pallas as pl
from jax.experimental.pallas import tpu as pltpu
from jax.experimental.pallas import tpu_sc as plsc

N = 10000
N_PAD = 10240
E = 160000
E_PAD = 163840
ROWS = E_PAD // 128
ROWS_PER_TILE = ROWS // 16
ACC_ROWS_PER_TILE = N_PAD // 16
F = 128
HW = 64
NBUF = 2


def _sc_body(with_deg, t0, t1, srcr, dstr, zros, acc0_out, acc1_out, *rest):
  if with_deg:
    acc, srcv, dstv = rest[1:4]
    deg_out = rest[0]
    bufs = rest[4:4 + NBUF]
    degv = rest[4 + NBUF]
    gsems = rest[5 + NBUF:]
  else:
    acc, srcv, dstv = rest[:3]
    bufs = rest[3:3 + NBUF]
    gsems = rest[3 + NBUF:]
    degv = None
  c = lax.axis_index("c")
  s = lax.axis_index("s")

  ibase = s * ROWS_PER_TILE
  pltpu.sync_copy(srcr.at[pl.ds(ibase, ROWS_PER_TILE)], srcv)
  pltpu.sync_copy(dstr.at[pl.ds(ibase, ROWS_PER_TILE)], dstv)

  rbase = s * ACC_ROWS_PER_TILE
  pltpu.sync_copy(zros.at[pl.ds(rbase, ACC_ROWS_PER_TILE)],
                  acc.at[pl.ds(rbase, ACC_ROWS_PER_TILE)])
  if with_deg:
    zeros16 = jnp.zeros((16,), jnp.float32)

    @pl.when(c == 0)
    def _():
      def zstep(j, carry):
        degv[pl.ds(j * 16, 16)] = zeros16
        return carry
      lax.fori_loop(0, N_PAD // 16, zstep, 0)

  plsc.subcore_barrier()

  ones16 = jnp.ones((16,), jnp.float32)

  def _process(table, count_deg):
    for b in range(NBUF):
      pltpu.async_copy(table.at[srcv.at[0, pl.ds(b * HW, HW)]], bufs[b],
                       gsems[b])

    def outer(i, carry):
      for b in range(NBUF):
        sl = pl.ds(b * HW, HW)
        pltpu.make_async_copy(table.at[srcv.at[i, sl]], bufs[b],
                              gsems[b]).wait()
        pltpu.sync_copy(bufs[b], acc.at[dstv.at[i, sl]], add=True)
        if count_deg:
          for k in range(HW // 16):
            idx = dstv[i, pl.ds(b * HW + k * 16, 16)]
            plsc.addupdate_scatter(degv, [idx], ones16)

        @pl.when(i + 1 < ROWS_PER_TILE)
        def _():
          pltpu.async_copy(table.at[srcv.at[i + 1, pl.ds(b * HW, HW)]],
                           bufs[b], gsems[b])
      return carry
    lax.fori_loop(0, ROWS_PER_TILE, outer, 0)

  @pl.when(c == 0)
  def _():
    _process(t0, with_deg)

  @pl.when(c == 1)
  def _():
    _process(t1, False)

  plsc.subcore_barrier()

  sl = pl.ds(rbase, ACC_ROWS_PER_TILE)

  @pl.when(c == 0)
  def _():
    pltpu.sync_copy(acc.at[sl], acc0_out.at[sl])
    if with_deg:
      pltpu.sync_copy(degv, deg_out.at[s])

  @pl.when(c == 1)
  def _():
    pltpu.sync_copy(acc.at[sl], acc1_out.at[sl])


@functools.lru_cache(maxsize=None)
def _make_sc_seg(with_deg):
  mesh = plsc.VectorSubcoreMesh(core_axis_name="c", subcore_axis_name="s",
                                num_cores=2, num_subcores=16)
  out_type = [jax.ShapeDtypeStruct((N_PAD, F), jnp.float32),
              jax.ShapeDtypeStruct((N_PAD, F), jnp.float32)]
  if with_deg:
    out_type += [jax.ShapeDtypeStruct((16, N_PAD), jnp.float32)]
  scratch = [
      pltpu.VMEM_SHARED((N_PAD, F), jnp.float32),
      pltpu.VMEM((ROWS_PER_TILE, 128), jnp.int32),
      pltpu.VMEM((ROWS_PER_TILE, 128), jnp.int32),
  ] + [pltpu.VMEM((HW, F), jnp.float32) for _ in range(NBUF)]
  if with_deg:
    scratch += [pltpu.VMEM((N_PAD,), jnp.float32)]
  scratch += [pltpu.SemaphoreType.DMA] * NBUF
  return pl.kernel(
      functools.partial(_sc_body, with_deg),
      out_type=tuple(out_type),
      mesh=mesh,
      scratch_types=tuple(scratch),
      compiler_params=pltpu.CompilerParams(needs_layout_passes=False),
  )


def _dense1_body(agg0, agg1, degp, x_ref, w1l, w1r, b1, out):
  deg = jnp.sum(degp[...], axis=1)[:, None]
  r = 1.0 / jnp.maximum(deg, 1.0)
  h = jnp.dot(agg0[...] * r, w1l[0:128, :], preferred_element_type=jnp.float32)
  h += jnp.dot(agg1[...] * r, w1l[128:256, :], preferred_element_type=jnp.float32)
  h += jnp.dot(x_ref[...], w1r[...], preferred_element_type=jnp.float32)
  h = jnp.maximum(h + b1[...], 0.0)
  for q in range(4):
    out[q, :, :] = h[:, 128 * q:128 * (q + 1)]


def _dense2_body(a0, a1, a2, a3, degp, h1, w2l, w2r, b2, w3, b3, out):
  deg = jnp.sum(degp[...], axis=1)[:, None]
  r = 1.0 / jnp.maximum(deg, 1.0)
  aggs = (a0, a1, a2, a3)
  h = b2[...] + jnp.zeros((a0.shape[0], 512), jnp.float32)
  for q in range(4):
    h += jnp.dot(aggs[q][...] * r, w2l[128 * q:128 * (q + 1), :],
                 preferred_element_type=jnp.float32)
    h += jnp.dot(h1[q, :, :], w2r[128 * q:128 * (q + 1), :],
                 preferred_element_type=jnp.float32)
  h = jnp.maximum(h, 0.0)
  out[...] = jnp.dot(h, w3[...], preferred_element_type=jnp.float32) + b3[...]


_MB = 1000


def _dense1(agg0, agg1, degp, x, W1l, W1r, b1):
  grid = (N // _MB,)
  return pl.pallas_call(
      _dense1_body,
      grid=grid,
      in_specs=[
          pl.BlockSpec((_MB, F), lambda i: (i, 0)),
          pl.BlockSpec((_MB, F), lambda i: (i, 0)),
          pl.BlockSpec((_MB, 16), lambda i: (i, 0)),
          pl.BlockSpec((_MB, 256), lambda i: (i, 0)),
          pl.BlockSpec((256, 512), lambda i: (0, 0)),
          pl.BlockSpec((256, 512), lambda i: (0, 0)),
          pl.BlockSpec((1, 512), lambda i: (0, 0)),
      ],
      out_specs=pl.BlockSpec((4, _MB, 128), lambda i: (0, i, 0)),
      out_shape=jax.ShapeDtypeStruct((4, N, 128), jnp.float32),
  )(agg0, agg1, degp, x, W1l, W1r, b1)


def _dense2(a0, a1, a2, a3, degp, h1t, W2l, W2r, b2, W3, b3):
  grid = (N // _MB,)
  agg_spec = pl.BlockSpec((_MB, F), lambda i: (i, 0))
  return pl.pallas_call(
      _dense2_body,
      grid=grid,
      in_specs=[
          agg_spec, agg_spec, agg_spec, agg_spec,
          pl.BlockSpec((_MB, 16), lambda i: (i, 0)),
          pl.BlockSpec((4, _MB, 128), lambda i: (0, i, 0)),
          pl.BlockSpec((512, 512), lambda i: (0, 0)),
          pl.BlockSpec((512, 512), lambda i: (0, 0)),
          pl.BlockSpec((1, 512), lambda i: (0, 0)),
          pl.BlockSpec((512, 256), lambda i: (0, 0)),
          pl.BlockSpec((1, 256), lambda i: (0, 0)),
      ],
      out_specs=pl.BlockSpec((_MB, 256), lambda i: (i, 0)),
      out_shape=jax.ShapeDtypeStruct((N, 256), jnp.float32),
  )(a0, a1, a2, a3, degp, h1t, W2l, W2r, b2, W3, b3)


def kernel(x, edge_index, W1l, b1l, W1r, W2l, b2l, W2r, W3, b3):
  x = x.astype(jnp.float32)
  src = edge_index[0].astype(jnp.int32)
  dst = edge_index[1].astype(jnp.int32)
  pad = E_PAD - E
  src2d = jnp.concatenate([src, jnp.zeros((pad,), jnp.int32)]).reshape(ROWS, 128)
  dst2d = jnp.concatenate([dst, jnp.full((pad,), N, jnp.int32)]).reshape(ROWS, 128)
  zeros = jnp.zeros((N_PAD, 128), jnp.float32)

  t0 = x[:, :128]
  t1 = x[:, 128:]
  agg0, agg1, degp = _make_sc_seg(True)(t0, t1, src2d, dst2d, zeros)
  degp = degp.T

  h1t = _dense1(agg0, agg1, degp, x, W1l, W1r, b1l.reshape(1, 512))

  a0, a1 = _make_sc_seg(False)(h1t[0], h1t[1], src2d, dst2d, zeros)
  a2, a3 = _make_sc_seg(False)(h1t[2], h1t[3], src2d, dst2d, zeros)

  return _dense2(a0, a1, a2, a3, degp, h1t,
                 W2l, W2r, b2l.reshape(1, 512), W3, b3.reshape(1, 256))
